# baseline (speedup 1.0000x reference)
import jax
import jax.numpy as jnp
from jax.experimental import pallas as pl

B, L, D = 128, 49, 768
NT = 196
NC = 1000
KEEP_RATIO = (0.75, 0.5, 0.25)
CLF_W, TOK_W, DIST_W = 1.0, 2.0, 0.5

BB = 16
PH = 14


def _teacher_kernel(x_ref, w_ref, tok_ref, sum_ref):
    ph = pl.program_id(0)
    acc = jnp.zeros((B, D), jnp.float32)
    for pw in range(14):
        slab = x_ref[:, 0, :, pw, :, :].reshape(D, B)
        z = jnp.tanh(
            jax.lax.dot_general(slab.astype(jnp.bfloat16), w_ref[...],
                                (((0,), (0,)), ((), ())),
                                preferred_element_type=jnp.float32))
        tok_ref[pw] = z.astype(jnp.bfloat16)
        acc = acc + z

    @pl.when(ph == 0)
    def _():
        sum_ref[...] = acc

    @pl.when(ph > 0)
    def _():
        sum_ref[...] = sum_ref[...] + acc


def _distill_kernel(g_ref, s_ref, pol_ref, klm_ref, msk_ref):
    mask = (pol_ref[...] > 0.5).astype(jnp.float32)
    msk_ref[...] = mask
    t = g_ref[...].astype(jnp.float32)
    s = s_ref[...]
    mt = jnp.max(t, axis=2, keepdims=True)
    et = jnp.exp(t - mt)
    st = jnp.sum(et, axis=2, keepdims=True)
    lse_t = mt + jnp.log(st)
    ms = jnp.max(s, axis=2, keepdims=True)
    lse_s = ms + jnp.log(jnp.sum(jnp.exp(s - ms), axis=2, keepdims=True))
    kl = (jnp.sum((et / st) * (t - s), axis=2)
          - lse_t[:, :, 0] + lse_s[:, :, 0])
    klm_ref[...] = kl * mask


def _losses_kernel(sum_ref, wct_ref, clst_ref, lab_ref, klm_ref, msk_ref,
                   pd0_ref, pd1_ref, pd2_ref,
                   loss_ref, clso_ref, ratio_ref, klo_ref, dist_ref):
    def lse0(x):
        m = jnp.max(x, axis=0, keepdims=True)
        return m + jnp.log(jnp.sum(jnp.exp(x - m), axis=0, keepdims=True))

    mean = sum_ref[...] * (1.0 / NT)
    tct = jax.lax.dot_general(wct_ref[...], mean, (((1,), (1,)), ((), ())),
                              preferred_element_type=jnp.float32)
    clst = clst_ref[...]
    log_t = tct - lse0(tct)
    log_s = clst - lse0(clst)
    kl_loss = jnp.sum(jnp.exp(log_t) * (log_t - log_s)) / B

    iota = jax.lax.broadcasted_iota(jnp.int32, (NC, B), 0)
    oh = (lab_ref[...] == iota).astype(jnp.float32)
    ce = -jnp.sum(log_s * oh) / B

    ratio = 0.0
    for r, pd in zip(KEEP_RATIO, (pd0_ref, pd1_ref, pd2_ref)):
        m = jnp.mean(pd[...], axis=0)
        ratio = ratio + jnp.mean((m - r) ** 2)

    nsel = jnp.sum(msk_ref[...])
    msum = jnp.sum(klm_ref[...])
    dist = jnp.where(nsel < 0.1, 0.0, msum / jnp.maximum(nsel, 1.0))

    cls_term = CLF_W * ce
    ratio_term = TOK_W * ratio / 3.0
    kl_term = DIST_W * kl_loss
    dist_term = DIST_W * dist
    loss_ref[...] = jnp.full((1, 1), cls_term + ratio_term + kl_term + dist_term,
                             jnp.float32)
    clso_ref[...] = jnp.full((1, 1), cls_term, jnp.float32)
    ratio_ref[...] = jnp.full((1, 1), ratio_term, jnp.float32)
    klo_ref[...] = jnp.full((1, 1), kl_term, jnp.float32)
    dist_ref[...] = jnp.full((1, 1), dist_term, jnp.float32)


@jax.jit
def kernel(inputs, cls_t, other_t, policy, pred_dec_0, pred_dec_1, pred_dec_2,
           current_pos, labels, W_patch, W_cls):
    x6 = inputs.transpose(1, 2, 3, 0).reshape(3, 14, 16, 14, 16, B)
    wct = W_cls.transpose(1, 0)
    clst = cls_t.transpose(1, 0)
    pdt = [p.transpose(1, 0) for p in (pred_dec_0, pred_dec_1, pred_dec_2)]
    wp_bf = W_patch.astype(jnp.bfloat16)

    tokens, token_sum = pl.pallas_call(
        _teacher_kernel,
        grid=(PH,),
        in_specs=[
            pl.BlockSpec((3, 1, 16, 14, 16, B), lambda i: (0, i, 0, 0, 0, 0)),
            pl.BlockSpec((D, D), lambda i: (0, 0)),
        ],
        out_specs=[
            pl.BlockSpec((14, B, D), lambda i: (i, 0, 0)),
            pl.BlockSpec((B, D), lambda i: (0, 0)),
        ],
        out_shape=[
            jax.ShapeDtypeStruct((NT, B, D), jnp.bfloat16),
            jax.ShapeDtypeStruct((B, D), jnp.float32),
        ],
    )(x6, wp_bf)

    gathered = tokens[current_pos, jnp.arange(B)[:, None], :]

    kl_masked, mask = pl.pallas_call(
        _distill_kernel,
        grid=(B // BB,),
        in_specs=[
            pl.BlockSpec((BB, L, D), lambda i: (i, 0, 0)),
            pl.BlockSpec((BB, L, D), lambda i: (i, 0, 0)),
            pl.BlockSpec((BB, L), lambda i: (i, 0)),
        ],
        out_specs=[
            pl.BlockSpec((BB, L), lambda i: (i, 0)),
            pl.BlockSpec((BB, L), lambda i: (i, 0)),
        ],
        out_shape=[
            jax.ShapeDtypeStruct((B, L), jnp.float32),
            jax.ShapeDtypeStruct((B, L), jnp.float32),
        ],
    )(gathered, other_t, policy)

    outs = pl.pallas_call(
        _losses_kernel,
        out_shape=[jax.ShapeDtypeStruct((1, 1), jnp.float32)] * 5,
    )(token_sum, wct, clst, labels.reshape(1, B), kl_masked, mask,
      pdt[0], pdt[1], pdt[2])

    return tuple(o[0, 0] for o in outs)


# speedup vs baseline: 1.2016x; 1.2016x over previous
import jax
import jax.numpy as jnp
from jax.experimental import pallas as pl

B, L, D = 128, 49, 768
NT = 196
NC = 1000
KEEP_RATIO = (0.75, 0.5, 0.25)
CLF_W, TOK_W, DIST_W = 1.0, 2.0, 0.5

BB = 16
PH = 14


def _teacher_kernel(x_ref, w_ref, tok_ref, sum_ref):
    ph = pl.program_id(0)
    acc = jnp.zeros((B, D), jnp.float32)
    for pw in range(14):
        slab = x_ref[:, 0, :, pw, :, :].reshape(D, B)
        z = jnp.tanh(
            jax.lax.dot_general(slab.astype(jnp.bfloat16), w_ref[...],
                                (((0,), (0,)), ((), ())),
                                preferred_element_type=jnp.float32))
        tok_ref[pw] = z.astype(jnp.bfloat16)
        acc = acc + z

    @pl.when(ph == 0)
    def _():
        sum_ref[...] = acc

    @pl.when(ph > 0)
    def _():
        sum_ref[...] = sum_ref[...] + acc


def _distill_kernel(g_ref, s_ref, pol_ref, klm_ref, msk_ref):
    mask = (pol_ref[...] > 0.5).astype(jnp.float32)
    msk_ref[...] = mask
    t = g_ref[...].astype(jnp.float32)
    s = s_ref[...]
    mt = jnp.max(t, axis=2, keepdims=True)
    et = jnp.exp(t - mt)
    st = jnp.sum(et, axis=2, keepdims=True)
    lse_t = mt + jnp.log(st)
    ms = jnp.max(s, axis=2, keepdims=True)
    lse_s = ms + jnp.log(jnp.sum(jnp.exp(s - ms), axis=2, keepdims=True))
    kl = (jnp.sum((et / st) * (t - s), axis=2)
          - lse_t[:, :, 0] + lse_s[:, :, 0])
    klm_ref[...] = kl * mask


def _losses_kernel(sum_ref, wct_ref, clst_ref, lab_ref, klm_ref, msk_ref,
                   pd0_ref, pd1_ref, pd2_ref,
                   loss_ref, clso_ref, ratio_ref, klo_ref, dist_ref):
    def lse0(x):
        m = jnp.max(x, axis=0, keepdims=True)
        return m + jnp.log(jnp.sum(jnp.exp(x - m), axis=0, keepdims=True))

    mean = sum_ref[...] * (1.0 / NT)
    tct = jax.lax.dot_general(wct_ref[...], mean, (((1,), (1,)), ((), ())),
                              preferred_element_type=jnp.float32)
    clst = clst_ref[...]
    log_t = tct - lse0(tct)
    log_s = clst - lse0(clst)
    kl_loss = jnp.sum(jnp.exp(log_t) * (log_t - log_s)) / B

    iota = jax.lax.broadcasted_iota(jnp.int32, (NC, B), 0)
    oh = (lab_ref[...] == iota).astype(jnp.float32)
    ce = -jnp.sum(log_s * oh) / B

    ratio = 0.0
    for r, pd in zip(KEEP_RATIO, (pd0_ref, pd1_ref, pd2_ref)):
        m = jnp.mean(pd[...], axis=0)
        ratio = ratio + jnp.mean((m - r) ** 2)

    nsel = jnp.sum(msk_ref[...])
    msum = jnp.sum(klm_ref[...])
    dist = jnp.where(nsel < 0.1, 0.0, msum / jnp.maximum(nsel, 1.0))

    cls_term = CLF_W * ce
    ratio_term = TOK_W * ratio / 3.0
    kl_term = DIST_W * kl_loss
    dist_term = DIST_W * dist
    loss_ref[...] = jnp.full((1, 1), cls_term + ratio_term + kl_term + dist_term,
                             jnp.float32)
    clso_ref[...] = jnp.full((1, 1), cls_term, jnp.float32)
    ratio_ref[...] = jnp.full((1, 1), ratio_term, jnp.float32)
    klo_ref[...] = jnp.full((1, 1), kl_term, jnp.float32)
    dist_ref[...] = jnp.full((1, 1), dist_term, jnp.float32)


@jax.jit
def kernel(inputs, cls_t, other_t, policy, pred_dec_0, pred_dec_1, pred_dec_2,
           current_pos, labels, W_patch, W_cls):
    x6 = inputs.transpose(1, 2, 3, 0).reshape(3, 14, 16, 14, 16, B)
    wct = W_cls.transpose(1, 0)
    clst = cls_t.transpose(1, 0)
    pdt = [p.transpose(1, 0) for p in (pred_dec_0, pred_dec_1, pred_dec_2)]
    wp_bf = W_patch.astype(jnp.bfloat16)

    tokens, token_sum = pl.pallas_call(
        _teacher_kernel,
        grid=(PH,),
        in_specs=[
            pl.BlockSpec((3, 1, 16, 14, 16, B), lambda i: (0, i, 0, 0, 0, 0)),
            pl.BlockSpec((D, D), lambda i: (0, 0)),
        ],
        out_specs=[
            pl.BlockSpec((14, B, D), lambda i: (i, 0, 0)),
            pl.BlockSpec((B, D), lambda i: (0, 0)),
        ],
        out_shape=[
            jax.ShapeDtypeStruct((NT, B, D), jnp.bfloat16),
            jax.ShapeDtypeStruct((B, D), jnp.float32),
        ],
    )(x6, wp_bf)

    flat_idx = current_pos * B + jnp.arange(B, dtype=jnp.int32)[:, None]
    gathered = jnp.take(tokens.reshape(NT * B, D), flat_idx, axis=0)

    kl_masked, mask = pl.pallas_call(
        _distill_kernel,
        grid=(B // BB,),
        in_specs=[
            pl.BlockSpec((BB, L, D), lambda i: (i, 0, 0)),
            pl.BlockSpec((BB, L, D), lambda i: (i, 0, 0)),
            pl.BlockSpec((BB, L), lambda i: (i, 0)),
        ],
        out_specs=[
            pl.BlockSpec((BB, L), lambda i: (i, 0)),
            pl.BlockSpec((BB, L), lambda i: (i, 0)),
        ],
        out_shape=[
            jax.ShapeDtypeStruct((B, L), jnp.float32),
            jax.ShapeDtypeStruct((B, L), jnp.float32),
        ],
    )(gathered, other_t, policy)

    outs = pl.pallas_call(
        _losses_kernel,
        out_shape=[jax.ShapeDtypeStruct((1, 1), jnp.float32)] * 5,
    )(token_sum, wct, clst, labels.reshape(1, B), kl_masked, mask,
      pdt[0], pdt[1], pdt[2])

    return tuple(o[0, 0] for o in outs)
